# Initial kernel scaffold; baseline (speedup 1.0000x reference)
import jax
import jax.numpy as jnp
from jax.experimental import pallas as pl


def kernel(img):
    raise NotImplementedError("write your pallas kernel here")


# speedup vs baseline: 126.0570x; 126.0570x over previous
import jax
import jax.numpy as jnp
from jax.experimental import pallas as pl
from jax.experimental.pallas import tpu as pltpu


def _med3(a, b, c):
    mn = jnp.minimum(a, b)
    mx = jnp.maximum(a, b)
    return jnp.maximum(mn, jnp.minimum(mx, c))


def _median3x3_kernel(x_ref, o_ref):
    x = x_ref[0]
    h, w = x.shape

    up = pltpu.roll(x, 1, axis=0)
    dn = pltpu.roll(x, h - 1, axis=0)
    row = jax.lax.broadcasted_iota(jnp.int32, (h, w), 0)
    up = jnp.where(row == 0, dn, up)
    dn = jnp.where(row == h - 1, up, dn)

    mn = jnp.minimum(up, dn)
    mx = jnp.maximum(up, dn)
    lo = jnp.minimum(mn, x)
    hi = jnp.maximum(mx, x)
    mi = jnp.maximum(mn, jnp.minimum(mx, x))

    col = jax.lax.broadcasted_iota(jnp.int32, (h, w), 1)

    def hshifts(t):
        lt = pltpu.roll(t, 1, axis=1)
        rt = pltpu.roll(t, w - 1, axis=1)
        ltf = jnp.where(col == 0, rt, lt)
        rtf = jnp.where(col == w - 1, lt, rt)
        return ltf, rtf

    lo_l, lo_r = hshifts(lo)
    hi_l, hi_r = hshifts(hi)
    mi_l, mi_r = hshifts(mi)

    a = jnp.maximum(jnp.maximum(lo_l, lo), lo_r)
    c = jnp.minimum(jnp.minimum(hi_l, hi), hi_r)
    b = _med3(mi_l, mi, mi_r)
    o_ref[0] = _med3(a, b, c)


def kernel(img):
    B, C, H, W = img.shape
    x = img.reshape(B * C, H, W)
    out = pl.pallas_call(
        _median3x3_kernel,
        out_shape=jax.ShapeDtypeStruct((B * C, H, W), img.dtype),
        grid=(B * C,),
        in_specs=[pl.BlockSpec((1, H, W), lambda i: (i, 0, 0))],
        out_specs=pl.BlockSpec((1, H, W), lambda i: (i, 0, 0)),
        compiler_params=pltpu.CompilerParams(
            dimension_semantics=("parallel",),
        ),
        name="median3x3",
    )(x)
    return out.reshape(B, C, H, W)


# speedup vs baseline: 155.9843x; 1.2374x over previous
import jax
import jax.numpy as jnp
from jax.experimental import pallas as pl
from jax.experimental.pallas import tpu as pltpu

_R = 256


def _med3(a, b, c):
    mn = jnp.minimum(a, b)
    mx = jnp.maximum(a, b)
    return jnp.maximum(mn, jnp.minimum(mx, c))


def _median3x3_kernel(top_ref, x_ref, bot_ref, o_ref):
    rows = x_ref.shape[1]
    w = x_ref.shape[2]
    n = rows // 8
    h = pl.program_id(1)
    nh = pl.num_programs(1)

    pv_top = jnp.where(h == 0, top_ref[0, 1:2, :], top_ref[0, 7:8, :])
    nv_bot = jnp.where(h == nh - 1, bot_ref[0, 6:7, :], bot_ref[0, 0:1, :])

    for s in range(n):
        cur = x_ref[0, s * 8 : (s + 1) * 8, :]
        pv = pv_top if s == 0 else x_ref[0, s * 8 - 1 : s * 8, :]
        nv = nv_bot if s == n - 1 else x_ref[0, (s + 1) * 8 : (s + 1) * 8 + 1, :]

        up = jnp.concatenate([pv, cur[:7, :]], axis=0)
        dn = jnp.concatenate([cur[1:, :], nv], axis=0)

        mn = jnp.minimum(up, dn)
        mx = jnp.maximum(up, dn)
        lo = jnp.minimum(mn, cur)
        hi = jnp.maximum(mx, cur)
        mi = jnp.maximum(mn, jnp.minimum(mx, cur))

        def hshifts(t):
            lt = jnp.concatenate([t[:, 1:2], t[:, : w - 1]], axis=1)
            rt = jnp.concatenate([t[:, 1:], t[:, w - 2 : w - 1]], axis=1)
            return lt, rt

        lo_l, lo_r = hshifts(lo)
        hi_l, hi_r = hshifts(hi)
        mi_l, mi_r = hshifts(mi)

        a = jnp.maximum(jnp.maximum(lo_l, lo), lo_r)
        c = jnp.minimum(jnp.minimum(hi_l, hi), hi_r)
        b = _med3(mi_l, mi, mi_r)
        o_ref[0, s * 8 : (s + 1) * 8, :] = _med3(a, b, c)


def kernel(img):
    B, C, H, W = img.shape
    r = _R if H % _R == 0 else H
    rb = r // 8
    x = img.reshape(B * C, H, W)
    hb = H // 8

    out = pl.pallas_call(
        _median3x3_kernel,
        out_shape=jax.ShapeDtypeStruct((B * C, H, W), img.dtype),
        grid=(B * C, H // r),
        in_specs=[
            pl.BlockSpec(
                (1, 8, W), lambda i, h: (i, jnp.maximum(h * rb - 1, 0), 0)
            ),
            pl.BlockSpec((1, r, W), lambda i, h: (i, h, 0)),
            pl.BlockSpec(
                (1, 8, W),
                lambda i, h: (i, jnp.minimum(h * rb + rb, hb - 1), 0),
            ),
        ],
        out_specs=pl.BlockSpec((1, r, W), lambda i, h: (i, h, 0)),
        compiler_params=pltpu.CompilerParams(
            dimension_semantics=("parallel", "arbitrary"),
        ),
        name="median3x3",
    )(x, x, x)
    return out.reshape(B, C, H, W)
